# baseline (speedup 1.0000x reference)
import jax
import jax.numpy as jnp
from jax.experimental import pallas as pl
from jax.experimental.pallas import tpu as pltpu

_N = 1000
_NPAD = 1024
_K = 128
_PROJ = 1024
_HID = 256
_NCLS = 150
_THR = 0.7
_NEG = float("-inf")


def _transpose_col(col, n, dtype=jnp.float32):
    eye = (jax.lax.broadcasted_iota(jnp.int32, (n, n), 0)
           == jax.lax.broadcasted_iota(jnp.int32, (n, n), 1))
    zero = jnp.zeros((), dtype)
    if col.shape[0] == 1:
        return jnp.sum(jnp.where(eye, jnp.broadcast_to(col, (n, n)), zero),
                       axis=1, keepdims=True)
    return jnp.sum(jnp.where(eye, jnp.broadcast_to(col, (n, n)), zero),
                   axis=0, keepdims=True)


def _rpn_body(s_ref, feat_ref, boxes_ref,
              W1s_ref, b1s_ref, W2s_ref, b2s_ref,
              W1o_ref, b1o_ref, W2o_ref, b2o_ref,
              bs_out, bo_out, fs_out, fo_out, fa_out, vals_out,
              P_ref, rm_ref, rc_ref, vals_scr, subj_ref, obj_ref):
    f32 = jnp.float32
    i32 = jnp.int32
    s = s_ref[...]
    feat = feat_ref[...]

    h = jnp.maximum(jnp.dot(s, W1s_ref[...], preferred_element_type=f32)
                    + b1s_ref[...], 0.0)
    u = (jnp.dot(h, W2s_ref[...], preferred_element_type=f32)
         + b2s_ref[...]) * feat
    h2 = jnp.maximum(jnp.dot(s, W1o_ref[...], preferred_element_type=f32)
                     + b1o_ref[...], 0.0)
    v = (jnp.dot(h2, W2o_ref[...], preferred_element_type=f32)
         + b2o_ref[...]) * feat

    L = jax.lax.dot_general(u, v, (((1,), (1,)), ((), ())),
                            preferred_element_type=f32)

    row = jax.lax.broadcasted_iota(i32, (_NPAD, _NPAD), 0)
    col = jax.lax.broadcasted_iota(i32, (_NPAD, _NPAD), 1)
    valid = (row < _N) & (col < _N) & (row != col)
    sig = 1.0 / (1.0 + jnp.exp(-L))
    P = jnp.where(valid, jnp.where(col > row, sig, 0.5), _NEG)
    P_ref[...] = P.reshape(_NPAD, 8, 128)
    P3 = P.reshape(8, 128, _NPAD)
    rm = jnp.max(P3, axis=2)
    rm_ref[...] = rm
    colio_big = jax.lax.broadcasted_iota(i32, (8, 128, _NPAD), 2)
    BIG = jnp.int32(1 << 30)
    rc_ref[...] = jnp.min(jnp.where(P3 == rm[:, :, None], colio_big, BIG),
                          axis=2)

    flat8 = (jax.lax.broadcasted_iota(i32, (8, 128), 0) * 128
             + jax.lax.broadcasted_iota(i32, (8, 128), 1))
    colio3 = (jax.lax.broadcasted_iota(i32, (1, 8, 128), 1) * 128
              + jax.lax.broadcasted_iota(i32, (1, 8, 128), 2))
    lane128 = jax.lax.broadcasted_iota(i32, (1, _K), 1)

    def select(k, _):
        rm = rm_ref[...]
        rc = rc_ref[...]
        m_b = jnp.max(rm, axis=(0, 1), keepdims=True)
        r_b = jnp.min(jnp.where(rm == m_b, flat8, BIG), axis=(0, 1),
                      keepdims=True)
        rmask = flat8 == r_b
        c_b = jnp.min(jnp.where(rmask, rc, BIG), axis=(0, 1), keepdims=True)
        r = r_b[0, 0]
        slab = P_ref[pl.ds(r, 1)]
        slab2 = jnp.where(colio3 == c_b.reshape(1, 1, 1), _NEG, slab)
        P_ref[pl.ds(r, 1)] = slab2
        nm = jnp.max(slab2, axis=(0, 1, 2), keepdims=True).reshape(1, 1)
        nc = jnp.min(jnp.where(slab2 == nm.reshape(1, 1, 1), colio3, BIG),
                     axis=(0, 1, 2), keepdims=True).reshape(1, 1)
        rm_ref[...] = jnp.where(rmask, nm, rm)
        rc_ref[...] = jnp.where(rmask, nc, rc)
        vals_scr[...] = jnp.where(lane128 == k, m_b, vals_scr[...])
        subj_ref[...] = jnp.where(lane128 == k, r_b, subj_ref[...])
        obj_ref[...] = jnp.where(lane128 == k, c_b, obj_ref[...])
        return 0

    jax.lax.fori_loop(0, _K, select, 0)

    subj_c = _transpose_col(subj_ref[...], _K, i32)
    obj_c = _transpose_col(obj_ref[...], _K, i32)
    col1024 = jax.lax.broadcasted_iota(i32, (_K, _NPAD), 1)
    oh_s = (col1024 == subj_c).astype(f32)
    oh_o = (col1024 == obj_c).astype(f32)
    bs = jnp.dot(oh_s, boxes_ref[...], preferred_element_type=f32)
    bo = jnp.dot(oh_o, boxes_ref[...], preferred_element_type=f32)
    fso = jnp.dot(oh_s, feat, preferred_element_type=f32)
    foo = jnp.dot(oh_o, feat, preferred_element_type=f32)

    ux1 = jnp.minimum(bs[:, 0:1], bo[:, 0:1])
    uy1 = jnp.minimum(bs[:, 1:2], bo[:, 1:2])
    ux2 = jnp.maximum(bs[:, 2:3], bo[:, 2:3])
    uy2 = jnp.maximum(bs[:, 3:4], bo[:, 3:4])
    area = (ux2 - ux1) * (uy2 - uy1)
    x1r = _transpose_col(ux1, _K)
    y1r = _transpose_col(uy1, _K)
    x2r = _transpose_col(ux2, _K)
    y2r = _transpose_col(uy2, _K)
    ar = _transpose_col(area, _K)
    ltx = jnp.maximum(ux1, x1r)
    lty = jnp.maximum(uy1, y1r)
    rbx = jnp.minimum(ux2, x2r)
    rby = jnp.minimum(uy2, y2r)
    wx = jnp.maximum(rbx - ltx, 0.0)
    wy = jnp.maximum(rby - lty, 0.0)
    inter = wx * wy
    iou = inter / (area + ar - inter + 1e-9)
    over = iou > _THR

    kp = jnp.ones((1, _K), f32)
    for i in range(_K):
        ki = kp[:, i:i + 1]
        sup = over[i:i + 1, :] & (lane128 > i) & (ki > 0.0)
        kp = jnp.where(sup, 0.0, kp)

    kc = _transpose_col(kp, _K)
    bs_out[...] = bs * kc
    bo_out[...] = bo * kc
    fs_out[...] = fso * kc
    fo_out[...] = foo * kc
    fa_out[...] = (fso + foo) * 0.5 * kc
    vals_out[...] = vals_scr[...] * kp


def kernel(boxes, scores, features, W1s, b1s, W2s, b2s, W1o, b1o, W2o, b2o):
    f32 = jnp.float32
    s = jnp.zeros((_NPAD, _NCLS), f32).at[:_N, :].set(scores[:, :-1])
    feat = jnp.zeros((_NPAD, _PROJ), f32).at[:_N, :].set(features)
    boxes_p = jnp.zeros((_NPAD, 4), f32).at[:_N, :].set(boxes)

    out_shapes = (
        jax.ShapeDtypeStruct((_K, 4), f32),
        jax.ShapeDtypeStruct((_K, 4), f32),
        jax.ShapeDtypeStruct((_K, _PROJ), f32),
        jax.ShapeDtypeStruct((_K, _PROJ), f32),
        jax.ShapeDtypeStruct((_K, _PROJ), f32),
        jax.ShapeDtypeStruct((1, _K), f32),
    )
    scratch = [
        pltpu.VMEM((_NPAD, 8, 128), f32),
        pltpu.VMEM((8, 128), f32),
        pltpu.VMEM((8, 128), jnp.int32),
        pltpu.VMEM((1, _K), f32),
        pltpu.VMEM((1, _K), jnp.int32),
        pltpu.VMEM((1, _K), jnp.int32),
    ]

    bs, bo, fs, fo, fa, vals = pl.pallas_call(
        _rpn_body,
        out_shape=out_shapes,
        scratch_shapes=scratch,
    )(s, feat, boxes_p,
      W1s, b1s.reshape(1, _HID), W2s, b2s.reshape(1, _PROJ),
      W1o, b1o.reshape(1, _HID), W2o, b2o.reshape(1, _PROJ))

    box_pairs = jnp.stack([bs, bo], axis=1)
    feats = jnp.stack([fs, fo, fa], axis=1)
    return box_pairs, feats, vals.reshape(_K)


# speedup vs baseline: 1.2738x; 1.2738x over previous
import jax
import jax.numpy as jnp
from jax.experimental import pallas as pl
from jax.experimental.pallas import tpu as pltpu

_N = 1000
_NPAD = 1024
_K = 128
_PROJ = 1024
_HID = 256
_NCLS = 150
_THR = 0.7
_NEG = float("-inf")


def _transpose_col(col, n, dtype=jnp.float32):
    eye = (jax.lax.broadcasted_iota(jnp.int32, (n, n), 0)
           == jax.lax.broadcasted_iota(jnp.int32, (n, n), 1))
    zero = jnp.zeros((), dtype)
    if col.shape[0] == 1:
        return jnp.sum(jnp.where(eye, jnp.broadcast_to(col, (n, n)), zero),
                       axis=1, keepdims=True)
    return jnp.sum(jnp.where(eye, jnp.broadcast_to(col, (n, n)), zero),
                   axis=0, keepdims=True)


def _rpn_body(s_ref, feat_ref, boxes_ref,
              W1s_ref, b1s_ref, W2s_ref, b2s_ref,
              W1o_ref, b1o_ref, W2o_ref, b2o_ref,
              bs_out, bo_out, fs_out, fo_out, fa_out, vals_out,
              P_ref):
    f32 = jnp.float32
    i32 = jnp.int32
    s = s_ref[...]
    feat = feat_ref[...]

    h = jnp.maximum(jnp.dot(s, W1s_ref[...], preferred_element_type=f32)
                    + b1s_ref[...], 0.0)
    u = (jnp.dot(h, W2s_ref[...], preferred_element_type=f32)
         + b2s_ref[...]) * feat
    h2 = jnp.maximum(jnp.dot(s, W1o_ref[...], preferred_element_type=f32)
                     + b1o_ref[...], 0.0)
    v = (jnp.dot(h2, W2o_ref[...], preferred_element_type=f32)
         + b2o_ref[...]) * feat

    L = jax.lax.dot_general(u, v, (((1,), (1,)), ((), ())),
                            preferred_element_type=f32)

    row = jax.lax.broadcasted_iota(i32, (_NPAD, _NPAD), 0)
    col = jax.lax.broadcasted_iota(i32, (_NPAD, _NPAD), 1)
    valid = (row < _N) & (col < _N) & (row != col)
    sig = 1.0 / (1.0 + jnp.exp(-L))
    P = jnp.where(valid, jnp.where(col > row, sig, 0.5), _NEG)
    P_ref[...] = P.reshape(_NPAD, 8, 128)
    P3 = P.reshape(8, 128, _NPAD)
    rm0 = jnp.max(P3, axis=2)
    BIGF = jnp.float32(2e9)
    colio_big = jax.lax.broadcasted_iota(i32, (8, 128, _NPAD), 2).astype(f32)
    rc0 = jnp.min(jnp.where(P3 == rm0[:, :, None], colio_big, BIGF), axis=2)

    flat8 = (jax.lax.broadcasted_iota(i32, (8, 128), 0) * 128
             + jax.lax.broadcasted_iota(i32, (8, 128), 1)).astype(f32)
    colio3 = (jax.lax.broadcasted_iota(i32, (1, 8, 128), 1) * 128
              + jax.lax.broadcasted_iota(i32, (1, 8, 128), 2)).astype(f32)
    lane128 = jax.lax.broadcasted_iota(i32, (1, _K), 1)

    def select(k, carry):
        rm, rc, vals_c, subj_c_, obj_c_ = carry
        m_b = jnp.max(rm, axis=(0, 1), keepdims=True)
        r_b = jnp.min(jnp.where(rm == m_b, flat8, BIGF), axis=(0, 1),
                      keepdims=True)
        rmask = flat8 == r_b
        c_b = jnp.min(jnp.where(rmask, rc, BIGF), axis=(0, 1), keepdims=True)
        r = r_b[0, 0].astype(i32)
        slab = P_ref[pl.ds(r, 1)]
        slab2 = jnp.where(colio3 == c_b.reshape(1, 1, 1), _NEG, slab)
        P_ref[pl.ds(r, 1)] = slab2
        nm = jnp.max(slab2, axis=(0, 1, 2), keepdims=True).reshape(1, 1)
        nc = jnp.min(jnp.where(slab2 == nm.reshape(1, 1, 1), colio3, BIGF),
                     axis=(0, 1, 2), keepdims=True).reshape(1, 1)
        sel = lane128 == k
        return (jnp.where(rmask, nm, rm),
                jnp.where(rmask, nc, rc),
                jnp.where(sel, m_b, vals_c),
                jnp.where(sel, r_b, subj_c_),
                jnp.where(sel, c_b, obj_c_))

    zrow = jnp.zeros((1, _K), f32)
    rm_f, rc_f, vals_v, subj_v, obj_v = jax.lax.fori_loop(
        0, _K, select, (rm0, rc0, zrow, zrow, zrow))

    subj_c = _transpose_col(subj_v, _K)
    obj_c = _transpose_col(obj_v, _K)
    col1024 = jax.lax.broadcasted_iota(i32, (_K, _NPAD), 1).astype(f32)
    oh_s = (col1024 == subj_c).astype(f32)
    oh_o = (col1024 == obj_c).astype(f32)
    hi = jax.lax.Precision.HIGHEST
    bs = jnp.dot(oh_s, boxes_ref[...], preferred_element_type=f32,
                 precision=hi)
    bo = jnp.dot(oh_o, boxes_ref[...], preferred_element_type=f32,
                 precision=hi)
    fso = jnp.dot(oh_s, feat, preferred_element_type=f32, precision=hi)
    foo = jnp.dot(oh_o, feat, preferred_element_type=f32, precision=hi)

    ux1 = jnp.minimum(bs[:, 0:1], bo[:, 0:1])
    uy1 = jnp.minimum(bs[:, 1:2], bo[:, 1:2])
    ux2 = jnp.maximum(bs[:, 2:3], bo[:, 2:3])
    uy2 = jnp.maximum(bs[:, 3:4], bo[:, 3:4])
    area = (ux2 - ux1) * (uy2 - uy1)
    x1r = _transpose_col(ux1, _K)
    y1r = _transpose_col(uy1, _K)
    x2r = _transpose_col(ux2, _K)
    y2r = _transpose_col(uy2, _K)
    ar = _transpose_col(area, _K)
    ltx = jnp.maximum(ux1, x1r)
    lty = jnp.maximum(uy1, y1r)
    rbx = jnp.minimum(ux2, x2r)
    rby = jnp.minimum(uy2, y2r)
    wx = jnp.maximum(rbx - ltx, 0.0)
    wy = jnp.maximum(rby - lty, 0.0)
    inter = wx * wy
    iou = inter / (area + ar - inter + 1e-9)
    over = iou > _THR

    kp = jnp.ones((1, _K), f32)
    for i in range(_K):
        ki = kp[:, i:i + 1]
        sup = over[i:i + 1, :] & (lane128 > i) & (ki > 0.0)
        kp = jnp.where(sup, 0.0, kp)

    kc = _transpose_col(kp, _K)
    bs_out[...] = bs * kc
    bo_out[...] = bo * kc
    fs_out[...] = fso * kc
    fo_out[...] = foo * kc
    fa_out[...] = (fso + foo) * 0.5 * kc
    vals_out[...] = vals_v * kp


def kernel(boxes, scores, features, W1s, b1s, W2s, b2s, W1o, b1o, W2o, b2o):
    f32 = jnp.float32
    s = jnp.zeros((_NPAD, _NCLS), f32).at[:_N, :].set(scores[:, :-1])
    feat = jnp.zeros((_NPAD, _PROJ), f32).at[:_N, :].set(features)
    boxes_p = jnp.zeros((_NPAD, 4), f32).at[:_N, :].set(boxes)

    out_shapes = (
        jax.ShapeDtypeStruct((_K, 4), f32),
        jax.ShapeDtypeStruct((_K, 4), f32),
        jax.ShapeDtypeStruct((_K, _PROJ), f32),
        jax.ShapeDtypeStruct((_K, _PROJ), f32),
        jax.ShapeDtypeStruct((_K, _PROJ), f32),
        jax.ShapeDtypeStruct((1, _K), f32),
    )
    scratch = [
        pltpu.VMEM((_NPAD, 8, 128), f32),
    ]

    bs, bo, fs, fo, fa, vals = pl.pallas_call(
        _rpn_body,
        out_shape=out_shapes,
        scratch_shapes=scratch,
    )(s, feat, boxes_p,
      W1s, b1s.reshape(1, _HID), W2s, b2s.reshape(1, _PROJ),
      W1o, b1o.reshape(1, _HID), W2o, b2o.reshape(1, _PROJ))

    box_pairs = jnp.stack([bs, bo], axis=1)
    feats = jnp.stack([fs, fo, fa], axis=1)
    return box_pairs, feats, vals.reshape(_K)


# speedup vs baseline: 1.4791x; 1.1612x over previous
import jax
import jax.numpy as jnp
from jax.experimental import pallas as pl
from jax.experimental.pallas import tpu as pltpu

_N = 1000
_NPAD = 1024
_K = 128
_PROJ = 1024
_HID = 256
_NCLS = 150
_THR = 0.7
_NEG = float("-inf")


def _transpose_col(col, n, dtype=jnp.float32):
    eye = (jax.lax.broadcasted_iota(jnp.int32, (n, n), 0)
           == jax.lax.broadcasted_iota(jnp.int32, (n, n), 1))
    zero = jnp.zeros((), dtype)
    if col.shape[0] == 1:
        return jnp.sum(jnp.where(eye, jnp.broadcast_to(col, (n, n)), zero),
                       axis=1, keepdims=True)
    return jnp.sum(jnp.where(eye, jnp.broadcast_to(col, (n, n)), zero),
                   axis=0, keepdims=True)


def _rpn_body(s_ref, feat_ref, boxes_ref,
              W1s_ref, b1s_ref, W2s_ref, b2s_ref,
              W1o_ref, b1o_ref, W2o_ref, b2o_ref,
              bs_out, bo_out, fs_out, fo_out, fa_out, vals_out,
              P_ref):
    f32 = jnp.float32
    i32 = jnp.int32
    s = s_ref[...]
    feat = feat_ref[...]

    h = jnp.maximum(jnp.dot(s, W1s_ref[...], preferred_element_type=f32)
                    + b1s_ref[...], 0.0)
    u = (jnp.dot(h, W2s_ref[...], preferred_element_type=f32)
         + b2s_ref[...]) * feat
    h2 = jnp.maximum(jnp.dot(s, W1o_ref[...], preferred_element_type=f32)
                     + b1o_ref[...], 0.0)
    v = (jnp.dot(h2, W2o_ref[...], preferred_element_type=f32)
         + b2o_ref[...]) * feat

    L = jax.lax.dot_general(u, v, (((1,), (1,)), ((), ())),
                            preferred_element_type=f32)

    row = jax.lax.broadcasted_iota(i32, (_NPAD, _NPAD), 0)
    col = jax.lax.broadcasted_iota(i32, (_NPAD, _NPAD), 1)
    valid = (row < _N) & (col < _N) & (row != col)
    sig = 1.0 / (1.0 + jnp.exp(-L))
    P = jnp.where(valid, jnp.where(col > row, sig, 0.5), _NEG)
    P_ref[...] = P.reshape(_NPAD, 8, 128)
    P3 = P.reshape(8, 128, _NPAD)
    rm0 = jnp.max(P3, axis=2)
    BIGF = jnp.float32(2e9)
    colio_big = jax.lax.broadcasted_iota(i32, (8, 128, _NPAD), 2).astype(f32)
    rc0 = jnp.min(jnp.where(P3 == rm0[:, :, None], colio_big, BIGF), axis=2)

    flat8 = (jax.lax.broadcasted_iota(i32, (8, 128), 0) * 128
             + jax.lax.broadcasted_iota(i32, (8, 128), 1)).astype(f32)
    colio3 = (jax.lax.broadcasted_iota(i32, (1, 8, 128), 1) * 128
              + jax.lax.broadcasted_iota(i32, (1, 8, 128), 2)).astype(f32)
    lane128 = jax.lax.broadcasted_iota(i32, (1, _K), 1)

    def select(k, carry):
        rm, rc, vals_c, subj_c_, obj_c_ = carry
        m_b = jnp.max(rm, axis=(0, 1), keepdims=True)
        r_b = jnp.min(jnp.where(rm == m_b, flat8, BIGF), axis=(0, 1),
                      keepdims=True)
        rmask = flat8 == r_b
        c_b = jnp.min(jnp.where(rmask, rc, BIGF), axis=(0, 1), keepdims=True)
        r = r_b[0, 0].astype(i32)
        slab = P_ref[pl.ds(r, 1)]
        slab2 = jnp.where(colio3 == c_b.reshape(1, 1, 1), _NEG, slab)
        P_ref[pl.ds(r, 1)] = slab2
        nm = jnp.max(slab2, axis=(0, 1, 2), keepdims=True).reshape(1, 1)
        nc = jnp.min(jnp.where(slab2 == nm.reshape(1, 1, 1), colio3, BIGF),
                     axis=(0, 1, 2), keepdims=True).reshape(1, 1)
        sel = lane128 == k
        return (jnp.where(rmask, nm, rm),
                jnp.where(rmask, nc, rc),
                jnp.where(sel, m_b, vals_c),
                jnp.where(sel, r_b, subj_c_),
                jnp.where(sel, c_b, obj_c_))

    zrow = jnp.zeros((1, _K), f32)
    carry = (rm0, rc0, zrow, zrow, zrow)
    for k in range(_K):
        carry = select(k, carry)
    rm_f, rc_f, vals_v, subj_v, obj_v = carry

    subj_c = _transpose_col(subj_v, _K)
    obj_c = _transpose_col(obj_v, _K)
    col1024 = jax.lax.broadcasted_iota(i32, (_K, _NPAD), 1).astype(f32)
    oh_s = (col1024 == subj_c).astype(f32)
    oh_o = (col1024 == obj_c).astype(f32)
    hi = jax.lax.Precision.HIGHEST
    bs = jnp.dot(oh_s, boxes_ref[...], preferred_element_type=f32,
                 precision=hi)
    bo = jnp.dot(oh_o, boxes_ref[...], preferred_element_type=f32,
                 precision=hi)
    fso = jnp.dot(oh_s, feat, preferred_element_type=f32, precision=hi)
    foo = jnp.dot(oh_o, feat, preferred_element_type=f32, precision=hi)

    ux1 = jnp.minimum(bs[:, 0:1], bo[:, 0:1])
    uy1 = jnp.minimum(bs[:, 1:2], bo[:, 1:2])
    ux2 = jnp.maximum(bs[:, 2:3], bo[:, 2:3])
    uy2 = jnp.maximum(bs[:, 3:4], bo[:, 3:4])
    area = (ux2 - ux1) * (uy2 - uy1)
    x1r = _transpose_col(ux1, _K)
    y1r = _transpose_col(uy1, _K)
    x2r = _transpose_col(ux2, _K)
    y2r = _transpose_col(uy2, _K)
    ar = _transpose_col(area, _K)
    ltx = jnp.maximum(ux1, x1r)
    lty = jnp.maximum(uy1, y1r)
    rbx = jnp.minimum(ux2, x2r)
    rby = jnp.minimum(uy2, y2r)
    wx = jnp.maximum(rbx - ltx, 0.0)
    wy = jnp.maximum(rby - lty, 0.0)
    inter = wx * wy
    iou = inter / (area + ar - inter + 1e-9)
    over = iou > _THR

    kp = jnp.ones((1, _K), f32)
    for i in range(_K):
        ki = kp[:, i:i + 1]
        sup = over[i:i + 1, :] & (lane128 > i) & (ki > 0.0)
        kp = jnp.where(sup, 0.0, kp)

    kc = _transpose_col(kp, _K)
    bs_out[...] = bs * kc
    bo_out[...] = bo * kc
    fs_out[...] = fso * kc
    fo_out[...] = foo * kc
    fa_out[...] = (fso + foo) * 0.5 * kc
    vals_out[...] = vals_v * kp


def kernel(boxes, scores, features, W1s, b1s, W2s, b2s, W1o, b1o, W2o, b2o):
    f32 = jnp.float32
    s = jnp.zeros((_NPAD, _NCLS), f32).at[:_N, :].set(scores[:, :-1])
    feat = jnp.zeros((_NPAD, _PROJ), f32).at[:_N, :].set(features)
    boxes_p = jnp.zeros((_NPAD, 4), f32).at[:_N, :].set(boxes)

    out_shapes = (
        jax.ShapeDtypeStruct((_K, 4), f32),
        jax.ShapeDtypeStruct((_K, 4), f32),
        jax.ShapeDtypeStruct((_K, _PROJ), f32),
        jax.ShapeDtypeStruct((_K, _PROJ), f32),
        jax.ShapeDtypeStruct((_K, _PROJ), f32),
        jax.ShapeDtypeStruct((1, _K), f32),
    )
    scratch = [
        pltpu.VMEM((_NPAD, 8, 128), f32),
    ]

    bs, bo, fs, fo, fa, vals = pl.pallas_call(
        _rpn_body,
        out_shape=out_shapes,
        scratch_shapes=scratch,
    )(s, feat, boxes_p,
      W1s, b1s.reshape(1, _HID), W2s, b2s.reshape(1, _PROJ),
      W1o, b1o.reshape(1, _HID), W2o, b2o.reshape(1, _PROJ))

    box_pairs = jnp.stack([bs, bo], axis=1)
    feats = jnp.stack([fs, fo, fa], axis=1)
    return box_pairs, feats, vals.reshape(_K)
